# baseline (speedup 1.0000x reference)
import functools

import jax
import jax.numpy as jnp
from jax import lax
from jax.experimental import pallas as pl
from jax.experimental.pallas import tpu as pltpu
from jax.experimental.pallas import tpu_sc as plsc

_NUM_EMB = 1000000
_DIM = 32
_BATCH = 16384
_FIELDS = 26

_NC = 2
_NS = 16

_LOG_C = 14
_CBLK = 1 << _LOG_C
_LOG_Q = _LOG_C - 2
_QBLK = _CBLK // 4
_GRID = (_NUM_EMB + _CBLK - 1) // _CBLK
_S_ROWS = _GRID * _QBLK


def _detile_body(tt_ref, s_ref):
    t = jnp.transpose(tt_ref[...])
    parts = [t[a * _QBLK:(a + 1) * _QBLK, :] for a in range(4)]
    s_ref[...] = jnp.concatenate(parts, axis=1)


@functools.cache
def _build_detile():
    return pl.pallas_call(
        _detile_body,
        grid=(_GRID,),
        in_specs=[pl.BlockSpec((_DIM, _CBLK), lambda g: (0, g))],
        out_specs=pl.BlockSpec((_QBLK, 128), lambda g: (g, 0)),
        out_shape=jax.ShapeDtypeStruct((_S_ROWS, 128), jnp.float32),
    )


_N_UNITS = 4 * _FIELDS


@functools.cache
def _build_gather():
    mesh = plsc.VectorSubcoreMesh(
        core_axis_name="c", subcore_axis_name="s", num_cores=_NC, num_subcores=_NS
    )

    @functools.partial(
        pl.kernel,
        mesh=mesh,
        out_type=jax.ShapeDtypeStruct((_FIELDS * 4 * 128 * 8, 128), jnp.float32),
        scratch_types=[
            pltpu.VMEM((_FIELDS, 4, 128), jnp.int32),
            pltpu.VMEM((4, 128, _DIM), jnp.float32),
            pltpu.VMEM((4, _DIM, 128), jnp.float32),
            [pltpu.SemaphoreType.DMA] * 4,
            [pltpu.SemaphoreType.DMA] * 4,
        ],
        compiler_params=pltpu.CompilerParams(
            use_tc_tiling_on_sc=False, needs_layout_passes=False
        ),
    )
    def gather_kernel(
        xt3_hbm, s_hbm, o_hbm, idx_v, rows_v, tile_v, sems, semws
    ):
        wid = lax.axis_index("s") * _NC + lax.axis_index("c")
        bt0 = wid * 4
        lane = lax.iota(jnp.int32, 16)
        pltpu.sync_copy(xt3_hbm.at[:, pl.ds(bt0, 4)], idx_v)

        def tbody(fb, carry):
            f = fb // 4
            b = fb % 4
            for g in range(8):
                v = idx_v[f, b, pl.ds(g * 16, 16)]
                j = (
                    ((v >> _LOG_C) << _LOG_C)
                    | ((v & (_QBLK - 1)) << 2)
                    | ((v >> _LOG_Q) & 3)
                )
                idx_v[f, b, pl.ds(g * 16, 16)] = j
            return carry

        lax.fori_loop(0, _FIELDS * 4, tbody, 0)

        def fire(u, e, sem):
            b = u // _FIELDS
            f = u % _FIELDS
            return pltpu.async_copy(s_hbm.at[idx_v.at[f, b]], rows_v.at[e], sem)

        def drain(u, e, sem):
            b = u // _FIELDS
            f = u % _FIELDS
            pltpu.make_async_copy(
                s_hbm.at[idx_v.at[f, b]], rows_v.at[e], sem
            ).wait()

        def qbase(u, st):
            b = u // _FIELDS
            f = u % _FIELDS
            return (f * 4 + st) * 128 + bt0 + b

        def drain_writes(u, e, semw):
            for st in range(4):
                q = qbase(u, st)
                pltpu.make_async_copy(
                    tile_v.at[e, pl.ds(st * 8, 8)], o_hbm.at[pl.ds(q * 8, 8)], semw
                ).wait()

        def process(u, e, semw):
            @pl.when(u >= 4)
            def _():
                drain_writes(u - 4, e, semw)

            def rbody(rg, c2):
                for rr in range(16):
                    r = rg * 16 + rr
                    for half in range(2):
                        v = rows_v[e, r, pl.ds(half * 16, 16)]
                        plsc.store_scatter(
                            tile_v.at[e],
                            [lane + half * 16, jnp.full((16,), r, jnp.int32)],
                            v,
                        )
                return c2

            lax.fori_loop(0, 8, rbody, 0)

            for st in range(4):
                q = qbase(u, st)
                pltpu.async_copy(
                    tile_v.at[e, pl.ds(st * 8, 8)], o_hbm.at[pl.ds(q * 8, 8)], semw
                )

        for e in range(4):
            fire(e, e, sems[e])

        def body(k, carry):
            u0 = 4 * k
            for e in range(4):
                u = u0 + e
                drain(u, e, sems[e])
                process(u, e, semws[e])

                @pl.when(u + 4 < _N_UNITS)
                def _():
                    fire(u + 4, e, sems[e])

            return carry

        lax.fori_loop(0, _N_UNITS // 4, body, 0)
        for e in range(4):
            drain_writes(_N_UNITS - 4 + e, e, semws[e])

    return gather_kernel


def kernel(x, embedding_weight):
    tt = jnp.transpose(embedding_weight)
    s = _build_detile()(tt)
    s_r = s.reshape(_S_ROWS * 4, _DIM)
    xt3 = jnp.transpose(x.astype(jnp.int32)).reshape(_FIELDS, 128, 128)
    o = _build_gather()(xt3, s_r)
    o5 = o.reshape(_FIELDS, 4, 128, 8, 128)
    o5t = jnp.transpose(o5, (2, 4, 0, 1, 3))
    return o5t.reshape(_BATCH, _FIELDS, _DIM)


# speedup vs baseline: 1.4167x; 1.4167x over previous
import functools

import jax
import jax.numpy as jnp
from jax import lax
from jax.experimental import pallas as pl
from jax.experimental.pallas import tpu as pltpu
from jax.experimental.pallas import tpu_sc as plsc

_NUM_EMB = 1000000
_DIM = 32
_BATCH = 16384
_FIELDS = 26

_NC = 2
_NS = 16

_LOG_C = 14
_CBLK = 1 << _LOG_C
_LOG_Q = _LOG_C - 2
_QBLK = _CBLK // 4
_GRID = (_NUM_EMB + _CBLK - 1) // _CBLK
_S_ROWS = _GRID * _QBLK


def _detile_body(tt_ref, s_ref):
    t = jnp.transpose(tt_ref[...])
    parts = [t[a * _QBLK:(a + 1) * _QBLK, :] for a in range(4)]
    s_ref[...] = jnp.concatenate(parts, axis=1)


@functools.cache
def _build_detile():
    return pl.pallas_call(
        _detile_body,
        grid=(_GRID,),
        in_specs=[pl.BlockSpec((_DIM, _CBLK), lambda g: (0, g))],
        out_specs=pl.BlockSpec((_QBLK, 128), lambda g: (g, 0)),
        out_shape=jax.ShapeDtypeStruct((_S_ROWS, 128), jnp.float32),
    )


_N_UNITS = 4 * _FIELDS


@functools.cache
def _build_gather():
    mesh = plsc.VectorSubcoreMesh(
        core_axis_name="c", subcore_axis_name="s", num_cores=_NC, num_subcores=_NS
    )

    @functools.partial(
        pl.kernel,
        mesh=mesh,
        out_type=jax.ShapeDtypeStruct((_FIELDS * 4 * 128 * 8, 128), jnp.float32),
        scratch_types=[
            pltpu.VMEM((_FIELDS, 4, 128), jnp.int32),
            pltpu.VMEM((4, 128, _DIM), jnp.float32),
            pltpu.VMEM((4, _DIM, 128), jnp.float32),
            [pltpu.SemaphoreType.DMA] * 4,
            [pltpu.SemaphoreType.DMA] * 4,
        ],
        compiler_params=pltpu.CompilerParams(
            use_tc_tiling_on_sc=False, needs_layout_passes=False
        ),
    )
    def gather_kernel(
        xt3_hbm, s_hbm, o_hbm, idx_v, rows_v, tile_v, sems, semws
    ):
        wid = lax.axis_index("s") * _NC + lax.axis_index("c")
        bt0 = wid * 4
        lane = lax.iota(jnp.int32, 16)
        pltpu.sync_copy(xt3_hbm.at[:, pl.ds(bt0, 4)], idx_v)

        def tbody(fb, carry):
            f = fb // 4
            b = fb % 4
            for g in range(8):
                v = idx_v[f, b, pl.ds(g * 16, 16)]
                j = (
                    ((v >> _LOG_C) << _LOG_C)
                    | ((v & (_QBLK - 1)) << 2)
                    | ((v >> _LOG_Q) & 3)
                )
                idx_v[f, b, pl.ds(g * 16, 16)] = j
            return carry

        lax.fori_loop(0, _FIELDS * 4, tbody, 0)

        def fire(u, e, sem):
            b = u // _FIELDS
            f = u % _FIELDS
            return pltpu.async_copy(s_hbm.at[idx_v.at[f, b]], rows_v.at[e], sem)

        def drain(u, e, sem):
            b = u // _FIELDS
            f = u % _FIELDS
            pltpu.make_async_copy(
                s_hbm.at[idx_v.at[f, b]], rows_v.at[e], sem
            ).wait()

        def qbase(u, st):
            b = u // _FIELDS
            f = u % _FIELDS
            return (f * 4 + st) * 128 + bt0 + b

        def drain_writes(u, e, semw):
            for st in range(4):
                q = qbase(u, st)
                pltpu.make_async_copy(
                    tile_v.at[e, pl.ds(st * 8, 8)], o_hbm.at[pl.ds(q * 8, 8)], semw
                ).wait()

        def process(u, e, semw):
            @pl.when(u >= 4)
            def _():
                drain_writes(u - 4, e, semw)

            def rbody(t, c2):
                c = (lane + t) & 15
                for half in range(2):
                    col = c + half * 16
                    for rb in range(8):
                        row = rb * 16 + lane
                        v = plsc.load_gather(rows_v.at[e], [row, col])
                        plsc.store_scatter(tile_v.at[e], [col, row], v)
                return c2

            lax.fori_loop(0, 16, rbody, 0)

            for st in range(4):
                q = qbase(u, st)
                pltpu.async_copy(
                    tile_v.at[e, pl.ds(st * 8, 8)], o_hbm.at[pl.ds(q * 8, 8)], semw
                )

        for e in range(4):
            fire(e, e, sems[e])

        def body(k, carry):
            u0 = 4 * k
            for e in range(4):
                u = u0 + e
                drain(u, e, sems[e])
                process(u, e, semws[e])

                @pl.when(u + 4 < _N_UNITS)
                def _():
                    fire(u + 4, e, sems[e])

            return carry

        lax.fori_loop(0, _N_UNITS // 4, body, 0)
        for e in range(4):
            drain_writes(_N_UNITS - 4 + e, e, semws[e])

    return gather_kernel


def kernel(x, embedding_weight):
    tt = jnp.transpose(embedding_weight)
    s = _build_detile()(tt)
    s_r = s.reshape(_S_ROWS * 4, _DIM)
    xt3 = jnp.transpose(x.astype(jnp.int32)).reshape(_FIELDS, 128, 128)
    o = _build_gather()(xt3, s_r)
    o5 = o.reshape(_FIELDS, 4, 128, 8, 128)
    o5t = jnp.transpose(o5, (2, 4, 0, 1, 3))
    return o5t.reshape(_BATCH, _FIELDS, _DIM)


# speedup vs baseline: 1.4174x; 1.0004x over previous
import functools

import jax
import jax.numpy as jnp
from jax import lax
from jax.experimental import pallas as pl
from jax.experimental.pallas import tpu as pltpu
from jax.experimental.pallas import tpu_sc as plsc

_NUM_EMB = 1000000
_DIM = 32
_BATCH = 16384
_FIELDS = 26

_NC = 2
_NS = 16

_LOG_C = 14
_CBLK = 1 << _LOG_C
_LOG_Q = _LOG_C - 2
_QBLK = _CBLK // 4
_GRID = (_NUM_EMB + _CBLK - 1) // _CBLK
_S_ROWS = _GRID * _QBLK


def _detile_body(tt_ref, s_ref):
    for a in range(4):
        t = jnp.transpose(tt_ref[:, a * _QBLK:(a + 1) * _QBLK])
        s_ref[:, a * _DIM:(a + 1) * _DIM] = t


@functools.cache
def _build_detile():
    return pl.pallas_call(
        _detile_body,
        grid=(_GRID,),
        in_specs=[pl.BlockSpec((_DIM, _CBLK), lambda g: (0, g))],
        out_specs=pl.BlockSpec((_QBLK, 128), lambda g: (g, 0)),
        out_shape=jax.ShapeDtypeStruct((_S_ROWS, 128), jnp.float32),
    )


_N_UNITS = 4 * _FIELDS


@functools.cache
def _build_gather():
    mesh = plsc.VectorSubcoreMesh(
        core_axis_name="c", subcore_axis_name="s", num_cores=_NC, num_subcores=_NS
    )

    @functools.partial(
        pl.kernel,
        mesh=mesh,
        out_type=jax.ShapeDtypeStruct((_FIELDS * 4 * 128 * 8, 128), jnp.float32),
        scratch_types=[
            pltpu.VMEM((_FIELDS, 4, 128), jnp.int32),
            pltpu.VMEM((4, 128, _DIM), jnp.float32),
            pltpu.VMEM((4, _DIM, 128), jnp.float32),
            [pltpu.SemaphoreType.DMA] * 4,
            [pltpu.SemaphoreType.DMA] * 4,
        ],
        compiler_params=pltpu.CompilerParams(
            use_tc_tiling_on_sc=False, needs_layout_passes=False
        ),
    )
    def gather_kernel(
        xt3_hbm, s_hbm, o_hbm, idx_v, rows_v, tile_v, sems, semws
    ):
        wid = lax.axis_index("s") * _NC + lax.axis_index("c")
        bt0 = wid * 4
        lane = lax.iota(jnp.int32, 16)
        pltpu.sync_copy(xt3_hbm.at[:, pl.ds(bt0, 4)], idx_v)

        def tbody(fb, carry):
            f = fb // 4
            b = fb % 4
            for g in range(8):
                v = idx_v[f, b, pl.ds(g * 16, 16)]
                j = (
                    ((v >> _LOG_C) << _LOG_C)
                    | ((v & (_QBLK - 1)) << 2)
                    | ((v >> _LOG_Q) & 3)
                )
                idx_v[f, b, pl.ds(g * 16, 16)] = j
            return carry

        lax.fori_loop(0, _FIELDS * 4, tbody, 0)

        def fire(u, e, sem):
            b = u // _FIELDS
            f = u % _FIELDS
            return pltpu.async_copy(s_hbm.at[idx_v.at[f, b]], rows_v.at[e], sem)

        def drain(u, e, sem):
            b = u // _FIELDS
            f = u % _FIELDS
            pltpu.make_async_copy(
                s_hbm.at[idx_v.at[f, b]], rows_v.at[e], sem
            ).wait()

        def qbase(u, st):
            b = u // _FIELDS
            f = u % _FIELDS
            return (f * 4 + st) * 128 + bt0 + b

        def drain_writes(u, e, semw):
            for st in range(4):
                q = qbase(u, st)
                pltpu.make_async_copy(
                    tile_v.at[e, pl.ds(st * 8, 8)], o_hbm.at[pl.ds(q * 8, 8)], semw
                ).wait()

        def process(u, e, semw):
            @pl.when(u >= 4)
            def _():
                drain_writes(u - 4, e, semw)

            def rbody(t, c2):
                c = (lane + t) & 15
                for half in range(2):
                    col = c + half * 16
                    for rb in range(8):
                        row = rb * 16 + lane
                        v = plsc.load_gather(rows_v.at[e], [row, col])
                        plsc.store_scatter(tile_v.at[e], [col, row], v)
                return c2

            lax.fori_loop(0, 16, rbody, 0)

            for st in range(4):
                q = qbase(u, st)
                pltpu.async_copy(
                    tile_v.at[e, pl.ds(st * 8, 8)], o_hbm.at[pl.ds(q * 8, 8)], semw
                )

        for e in range(4):
            fire(e, e, sems[e])

        def body(k, carry):
            u0 = 4 * k
            for e in range(4):
                u = u0 + e
                drain(u, e, sems[e])
                process(u, e, semws[e])

                @pl.when(u + 4 < _N_UNITS)
                def _():
                    fire(u + 4, e, sems[e])

            return carry

        lax.fori_loop(0, _N_UNITS // 4, body, 0)
        for e in range(4):
            drain_writes(_N_UNITS - 4 + e, e, semws[e])

    return gather_kernel


def kernel(x, embedding_weight):
    tt = jnp.transpose(embedding_weight)
    s = _build_detile()(tt)
    s_r = s.reshape(_S_ROWS * 4, _DIM)
    xt3 = jnp.transpose(x.astype(jnp.int32)).reshape(_FIELDS, 128, 128)
    o = _build_gather()(xt3, s_r)
    o5 = o.reshape(_FIELDS, 4, 128, 8, 128)
    o5t = jnp.transpose(o5, (2, 4, 0, 1, 3))
    return o5t.reshape(_BATCH, _FIELDS, _DIM)
